# baseline (speedup 1.0000x reference)
import functools

import jax
import jax.numpy as jnp
from jax import lax
from jax.experimental import pallas as pl
from jax.experimental.pallas import tpu as pltpu


def _round8(v):
    return -(-v // 8) * 8


def _fused_kernel(x_ref, wsh_ref, bsh_ref, w3f_ref, b3_ref, w5f_ref, b5_ref,
                  wmp_ref, bmp_ref, o_ref,
                  xpad, m1pad, ypad, p3, p5, yout,
                  *, P, H, W, cin, r3p, r5p, c1, c3, c5, cpool, c0):
    HW = H * W
    L = P * HW
    S = cin // 8
    f32 = jnp.float32
    bf16 = jnp.bfloat16
    NEG = jnp.array(-jnp.inf, bf16)

    for i in range(P):
        slab = x_ref[i].astype(bf16)
        for b in range(8):
            xpad[b * S:(b + 1) * S, c0 + i * HW:c0 + (i + 1) * HW] = (
                slab[:, b * HW:(b + 1) * HW])

    lane = lax.broadcasted_iota(jnp.int32, (1, L), 1)
    pos = lane % HW
    xc = pos % W
    yc = pos // W
    okx = {dx: (xc + dx >= 0) & (xc + dx < W) for dx in (-2, -1, 1, 2)}
    oky = {dy: (yc + dy >= 0) & (yc + dy < H) for dy in (-2, -1, 1, 2)}

    def tap_mask(dy, dx):
        if dy == 0 and dx == 0:
            return None
        if dy == 0:
            return okx[dx]
        if dx == 0:
            return oky[dy]
        return okx[dx] & oky[dy]

    xc_b = xpad[:, c0:c0 + L]
    y = jnp.dot(wsh_ref[...], xc_b, preferred_element_type=f32)
    y = jnp.maximum(y + bsh_ref[...], 0.0)

    rr = r3p + r5p
    yout[0:c1, :] = y[rr:rr + c1]

    ypad[:, c0:c0 + L] = y[0:rr]
    for dy in range(-1, 2):
        for dx in range(-1, 2):
            t = (dy + 1) * 3 + (dx + 1)
            s = dy * W + dx
            win = ypad[0:r3p, c0 + s:c0 + s + L]
            m = tap_mask(dy, dx)
            if m is not None:
                win = jnp.where(m, win, 0.0)
            p3[t * r3p:(t + 1) * r3p, :] = win
    for dy in range(-2, 3):
        for dx in range(-2, 3):
            t = (dy + 2) * 5 + (dx + 2)
            s = dy * W + dx
            win = ypad[r3p:rr, c0 + s:c0 + s + L]
            m = tap_mask(dy, dx)
            if m is not None:
                win = jnp.where(m, win, 0.0)
            p5[t * r5p:(t + 1) * r5p, :] = win

    o3 = jnp.dot(w3f_ref[...], p3[...], preferred_element_type=f32)
    o3 = jnp.maximum(o3 + b3_ref[...], 0.0)
    yout[c1:c1 + c3, :] = o3
    o5 = jnp.dot(w5f_ref[...], p5[...], preferred_element_type=f32)
    o5 = jnp.maximum(o5 + b5_ref[...], 0.0)
    yout[c1 + c3:c1 + c3 + c5, :] = o5

    wl = jnp.where(okx[-1], xpad[:, c0 - 1:c0 - 1 + L], NEG)
    wr = jnp.where(okx[1], xpad[:, c0 + 1:c0 + 1 + L], NEG)
    m1 = jnp.maximum(jnp.maximum(wl, wr), xc_b)
    m1pad[:, c0:c0 + L] = m1
    vu = jnp.where(oky[-1], m1pad[:, c0 - W:c0 - W + L], NEG)
    vd = jnp.where(oky[1], m1pad[:, c0 + W:c0 + W + L], NEG)
    pooled = jnp.maximum(jnp.maximum(vu, vd), m1)

    omp = jnp.dot(wmp_ref[...], pooled, preferred_element_type=f32)
    omp = jnp.maximum(omp + bmp_ref[...], 0.0)
    cc = c1 + c3 + c5
    yout[cc:cc + cpool, :] = omp

    for i in range(P):
        o_ref[i] = yout[:, i * HW:(i + 1) * HW]


def _inception_fused(x, w1, b1, w3r, b3r, w3, b3, w5r, b5r, w5, b5, wmp, bmp):
    N, cin, H, W = x.shape
    HW = H * W
    f32 = jnp.float32
    bf16 = jnp.bfloat16

    c1 = w1.shape[1]
    cr3 = w3r.shape[1]
    cr5 = w5r.shape[1]
    c3 = w3.shape[-1]
    c5 = w5.shape[-1]
    cpool = wmp.shape[1]
    ctot = c1 + c3 + c5 + cpool
    r3p = _round8(cr3)
    r5p = _round8(cr5)
    rr = r3p + r5p

    P = next(p for p in (16, 8, 4, 2, 1) if N % p == 0)
    L = P * HW
    c0 = 128

    S = cin // 8
    x3 = x.reshape(N, S, 8 * HW)
    r = jnp.arange(cin)
    perm = 8 * (r % S) + r // S

    z3 = jnp.zeros((r3p - cr3, cin), f32)
    z5 = jnp.zeros((r5p - cr5, cin), f32)
    wsh = jnp.concatenate([w3r.T, z3, w5r.T, z5, w1.T],
                          axis=0)[:, perm].astype(bf16)
    bsh = jnp.concatenate([b3r.T, z3[:, :1], b5r.T, z5[:, :1], b1.T], axis=0)

    w3p = jnp.pad(w3, ((0, 0), (0, 0), (0, r3p - cr3), (0, 0)))
    w3f = jnp.transpose(w3p, (3, 0, 1, 2)).reshape(c3, 9 * r3p)
    w5p = jnp.pad(w5, ((0, 0), (0, 0), (0, r5p - cr5), (0, 0)))
    w5f = jnp.transpose(w5p, (3, 0, 1, 2)).reshape(c5, 25 * r5p)

    kern = functools.partial(
        _fused_kernel, P=P, H=H, W=W, cin=cin, r3p=r3p, r5p=r5p,
        c1=c1, c3=c3, c5=c5, cpool=cpool, c0=c0)

    hwp = L + 2 * c0
    out = pl.pallas_call(
        kern,
        out_shape=jax.ShapeDtypeStruct((N, ctot, HW), f32),
        grid=(N // P,),
        in_specs=[
            pl.BlockSpec((P, S, 8 * HW), lambda g: (g, 0, 0)),
            pl.BlockSpec((rr + c1, cin), lambda g: (0, 0)),
            pl.BlockSpec((rr + c1, 1), lambda g: (0, 0)),
            pl.BlockSpec((c3, 9 * r3p), lambda g: (0, 0)),
            pl.BlockSpec((c3, 1), lambda g: (0, 0)),
            pl.BlockSpec((c5, 25 * r5p), lambda g: (0, 0)),
            pl.BlockSpec((c5, 1), lambda g: (0, 0)),
            pl.BlockSpec((cpool, cin), lambda g: (0, 0)),
            pl.BlockSpec((cpool, 1), lambda g: (0, 0)),
        ],
        out_specs=pl.BlockSpec((P, ctot, HW), lambda g: (g, 0, 0)),
        scratch_shapes=[
            pltpu.VMEM((cin, hwp), bf16),
            pltpu.VMEM((cin, hwp), bf16),
            pltpu.VMEM((rr, hwp), f32),
            pltpu.VMEM((9 * r3p, L), f32),
            pltpu.VMEM((25 * r5p, L), f32),
            pltpu.VMEM((ctot, L), f32),
        ],
        compiler_params=pltpu.CompilerParams(
            dimension_semantics=("parallel",),
            vmem_limit_bytes=64 << 20),
    )(x3, wsh, bsh, w3f, b3.T, w5f, b5.T,
      wmp.T[:, perm].astype(bf16), bmp.T)

    return out.reshape(N, ctot, H, W)


kernel = jax.jit(_inception_fused)


# speedup vs baseline: 1.1794x; 1.1794x over previous
import functools

import jax
import jax.numpy as jnp
from jax import lax
from jax.experimental import pallas as pl
from jax.experimental.pallas import tpu as pltpu

_TA = (((0,), (0,)), ((), ()))


def _fused_kernel(x_ref, w3r_ref, b3r_ref, w5r_ref, b5r_ref, w1_ref, b1_ref,
                  w3f_ref, b3_ref, w5f_ref, b5_ref, wmp_ref, bmp_ref, o_ref,
                  xpad, m1pad, ypad, p3, p5, yout,
                  *, P, H, W, cin, cr3, cr5, c1, c3, c5, cpool, c0):
    HW = H * W
    L = P * HW
    f32 = jnp.float32
    bf16 = jnp.bfloat16
    NEG = jnp.array(-jnp.inf, bf16)

    for i in range(P):
        xpad[:, c0 + i * HW:c0 + (i + 1) * HW] = x_ref[i].astype(bf16)

    lane = lax.broadcasted_iota(jnp.int32, (1, L), 1)
    pos = lane % HW
    xc = pos % W
    yc = pos // W
    okx = {dx: (xc + dx >= 0) & (xc + dx < W) for dx in (-2, -1, 1, 2)}
    oky = {dy: (yc + dy >= 0) & (yc + dy < H) for dy in (-2, -1, 1, 2)}

    def tap_mask(dy, dx):
        if dy == 0 and dx == 0:
            return None
        if dy == 0:
            return okx[dx]
        if dx == 0:
            return oky[dy]
        return okx[dx] & oky[dy]

    wsh = jnp.concatenate(
        [w3r_ref[...], w5r_ref[...], w1_ref[...]], axis=1).astype(bf16)
    bsh = jnp.concatenate(
        [b3r_ref[...], b5r_ref[...], b1_ref[...]], axis=1).T
    xc_b = xpad[:, c0:c0 + L]
    y = lax.dot_general(wsh, xc_b, _TA, preferred_element_type=f32)
    y = jnp.maximum(y + bsh, 0.0)

    rr = cr3 + cr5
    yout[0:c1, :] = y[rr:rr + c1]

    ypad[:, c0:c0 + L] = y[0:rr]
    for dy in range(-1, 2):
        for dx in range(-1, 2):
            t = (dy + 1) * 3 + (dx + 1)
            s = dy * W + dx
            win = ypad[0:cr3, c0 + s:c0 + s + L]
            m = tap_mask(dy, dx)
            if m is not None:
                win = jnp.where(m, win, 0.0)
            p3[t * cr3:(t + 1) * cr3, :] = win
    for dy in range(-2, 3):
        for dx in range(-2, 3):
            t = (dy + 2) * 5 + (dx + 2)
            s = dy * W + dx
            win = ypad[cr3:rr, c0 + s:c0 + s + L]
            m = tap_mask(dy, dx)
            if m is not None:
                win = jnp.where(m, win, 0.0)
            p5[t * cr5:(t + 1) * cr5, :] = win

    o3 = lax.dot_general(w3f_ref[...], p3[...], _TA, preferred_element_type=f32)
    o3 = jnp.maximum(o3 + b3_ref[...].T, 0.0)
    yout[c1:c1 + c3, :] = o3
    o5 = lax.dot_general(w5f_ref[...], p5[...], _TA, preferred_element_type=f32)
    o5 = jnp.maximum(o5 + b5_ref[...].T, 0.0)
    yout[c1 + c3:c1 + c3 + c5, :] = o5

    wl = jnp.where(okx[-1], xpad[:, c0 - 1:c0 - 1 + L], NEG)
    wr = jnp.where(okx[1], xpad[:, c0 + 1:c0 + 1 + L], NEG)
    m1 = jnp.maximum(jnp.maximum(wl, wr), xc_b)
    m1pad[:, c0:c0 + L] = m1
    vu = jnp.where(oky[-1], m1pad[:, c0 - W:c0 - W + L], NEG)
    vd = jnp.where(oky[1], m1pad[:, c0 + W:c0 + W + L], NEG)
    pooled = jnp.maximum(jnp.maximum(vu, vd), m1)

    omp = lax.dot_general(wmp_ref[...].astype(bf16), pooled, _TA,
                          preferred_element_type=f32)
    omp = jnp.maximum(omp + bmp_ref[...].T, 0.0)
    cc = c1 + c3 + c5
    yout[cc:cc + cpool, :] = omp

    for i in range(P):
        o_ref[i] = yout[:, i * HW:(i + 1) * HW]


def _inception_fused(x, w1, b1, w3r, b3r, w3, b3, w5r, b5r, w5, b5, wmp, bmp):
    N, cin, H, W = x.shape
    HW = H * W
    f32 = jnp.float32

    c1 = w1.shape[1]
    cr3 = w3r.shape[1]
    cr5 = w5r.shape[1]
    c3 = w3.shape[-1]
    c5 = w5.shape[-1]
    cpool = wmp.shape[1]
    ctot = c1 + c3 + c5 + cpool
    rr = cr3 + cr5

    P = next(p for p in (16, 8, 4, 2, 1) if N % p == 0)
    L = P * HW
    c0 = 128

    x3 = x.reshape(N, cin, HW)
    w3f = w3.reshape(9 * cr3, c3)
    w5f = w5.reshape(25 * cr5, c5)

    kern = functools.partial(
        _fused_kernel, P=P, H=H, W=W, cin=cin, cr3=cr3, cr5=cr5,
        c1=c1, c3=c3, c5=c5, cpool=cpool, c0=c0)

    hwp = L + 2 * c0
    full = lambda g: (0, 0)
    out = pl.pallas_call(
        kern,
        out_shape=jax.ShapeDtypeStruct((N, ctot, HW), f32),
        grid=(N // P,),
        in_specs=[
            pl.BlockSpec((P, cin, HW), lambda g: (g, 0, 0)),
            pl.BlockSpec((cin, cr3), full),
            pl.BlockSpec((1, cr3), full),
            pl.BlockSpec((cin, cr5), full),
            pl.BlockSpec((1, cr5), full),
            pl.BlockSpec((cin, c1), full),
            pl.BlockSpec((1, c1), full),
            pl.BlockSpec((9 * cr3, c3), full),
            pl.BlockSpec((1, c3), full),
            pl.BlockSpec((25 * cr5, c5), full),
            pl.BlockSpec((1, c5), full),
            pl.BlockSpec((cin, cpool), full),
            pl.BlockSpec((1, cpool), full),
        ],
        out_specs=pl.BlockSpec((P, ctot, HW), lambda g: (g, 0, 0)),
        scratch_shapes=[
            pltpu.VMEM((cin, hwp), jnp.bfloat16),
            pltpu.VMEM((cin, hwp), jnp.bfloat16),
            pltpu.VMEM((rr, hwp), f32),
            pltpu.VMEM((9 * cr3, L), f32),
            pltpu.VMEM((25 * cr5, L), f32),
            pltpu.VMEM((ctot, L), f32),
        ],
        compiler_params=pltpu.CompilerParams(
            dimension_semantics=("parallel",),
            vmem_limit_bytes=64 << 20),
    )(x3, w3r, b3r, w5r, b5r, w1, b1, w3f, b3, w5f, b5, wmp, bmp)

    return out.reshape(N, ctot, H, W)


kernel = jax.jit(_inception_fused)


# speedup vs baseline: 1.2078x; 1.0241x over previous
import functools

import jax
import jax.numpy as jnp
from jax import lax
from jax.experimental import pallas as pl
from jax.experimental.pallas import tpu as pltpu

_TA = (((0,), (0,)), ((), ()))


def _fused_kernel(x_ref, w3r_ref, b3r_ref, w5r_ref, b5r_ref, w1_ref, b1_ref,
                  w3f_ref, b3_ref, w5f_ref, b5_ref, wmp_ref, bmp_ref, o_ref,
                  xpad, m1pad, ypad, p3, p5, yout,
                  *, P, H, W, cin, cr3, cr5, c1, c3, c5, cpool, c0):
    HW = H * W
    L = P * HW
    f32 = jnp.float32
    bf16 = jnp.bfloat16
    NEG = jnp.array(-jnp.inf, bf16)

    for i in range(P):
        xpad[:, c0 + i * HW:c0 + (i + 1) * HW] = x_ref[i].astype(bf16)

    lane = lax.broadcasted_iota(jnp.int32, (1, L), 1)
    pos = lane % HW
    xc = pos % W
    yc = pos // W
    okx = {dx: (xc + dx >= 0) & (xc + dx < W) for dx in (-2, -1, 1, 2)}
    oky = {dy: (yc + dy >= 0) & (yc + dy < H) for dy in (-2, -1, 1, 2)}

    def tap_mask(dy, dx):
        if dy == 0 and dx == 0:
            return None
        if dy == 0:
            return okx[dx]
        if dx == 0:
            return oky[dy]
        return okx[dx] & oky[dy]

    wsh = jnp.concatenate(
        [w3r_ref[...], w5r_ref[...], w1_ref[...]], axis=1).astype(bf16)
    bsh = jnp.concatenate(
        [b3r_ref[...], b5r_ref[...], b1_ref[...]], axis=1).T
    xc_b = xpad[:, c0:c0 + L]
    y = lax.dot_general(wsh, xc_b, _TA, preferred_element_type=f32)
    y = jnp.maximum(y + bsh, 0.0)

    rr = cr3 + cr5
    yout[0:c1, :] = y[rr:rr + c1]

    ypad[:, c0:c0 + L] = y[0:rr]
    for dy in range(-1, 2):
        for dx in range(-1, 2):
            t = (dy + 1) * 3 + (dx + 1)
            s = dy * W + dx
            win = ypad[0:cr3, c0 + s:c0 + s + L]
            m = tap_mask(dy, dx)
            if m is not None:
                win = jnp.where(m, win, 0.0)
            p3[t * cr3:(t + 1) * cr3, :] = win
    for dy in range(-2, 3):
        for dx in range(-2, 3):
            t = (dy + 2) * 5 + (dx + 2)
            s = dy * W + dx
            win = ypad[cr3:rr, c0 + s:c0 + s + L]
            m = tap_mask(dy, dx)
            if m is not None:
                win = jnp.where(m, win, 0.0)
            p5[t * cr5:(t + 1) * cr5, :] = win

    o3 = lax.dot_general(w3f_ref[...], p3[...], _TA, preferred_element_type=f32)
    o3 = jnp.maximum(o3 + b3_ref[...].T, 0.0)
    yout[c1:c1 + c3, :] = o3
    o5 = lax.dot_general(w5f_ref[...], p5[...], _TA, preferred_element_type=f32)
    o5 = jnp.maximum(o5 + b5_ref[...].T, 0.0)
    yout[c1 + c3:c1 + c3 + c5, :] = o5

    wl = jnp.where(okx[-1], xpad[:, c0 - 1:c0 - 1 + L], NEG)
    wr = jnp.where(okx[1], xpad[:, c0 + 1:c0 + 1 + L], NEG)
    m1 = jnp.maximum(jnp.maximum(wl, wr), xc_b)
    m1pad[:, c0:c0 + L] = m1
    vu = jnp.where(oky[-1], m1pad[:, c0 - W:c0 - W + L], NEG)
    vd = jnp.where(oky[1], m1pad[:, c0 + W:c0 + W + L], NEG)
    pooled = jnp.maximum(jnp.maximum(vu, vd), m1)

    omp = lax.dot_general(wmp_ref[...].astype(bf16), pooled, _TA,
                          preferred_element_type=f32)
    omp = jnp.maximum(omp + bmp_ref[...].T, 0.0)
    cc = c1 + c3 + c5
    yout[cc:cc + cpool, :] = omp

    for i in range(P):
        o_ref[i] = yout[:, i * HW:(i + 1) * HW]


def _inception_fused(x, w1, b1, w3r, b3r, w3, b3, w5r, b5r, w5, b5, wmp, bmp):
    N, cin, H, W = x.shape
    HW = H * W
    f32 = jnp.float32

    c1 = w1.shape[1]
    cr3 = w3r.shape[1]
    cr5 = w5r.shape[1]
    c3 = w3.shape[-1]
    c5 = w5.shape[-1]
    cpool = wmp.shape[1]
    ctot = c1 + c3 + c5 + cpool
    rr = cr3 + cr5

    P = next(p for p in (32, 16, 8, 4, 2, 1) if N % p == 0)
    L = P * HW
    c0 = 128

    x3 = x.reshape(N, cin, HW)
    w3f = w3.reshape(9 * cr3, c3)
    w5f = w5.reshape(25 * cr5, c5)

    kern = functools.partial(
        _fused_kernel, P=P, H=H, W=W, cin=cin, cr3=cr3, cr5=cr5,
        c1=c1, c3=c3, c5=c5, cpool=cpool, c0=c0)

    hwp = L + 2 * c0
    full = lambda g: (0, 0)
    out = pl.pallas_call(
        kern,
        out_shape=jax.ShapeDtypeStruct((N, ctot, HW), f32),
        grid=(N // P,),
        in_specs=[
            pl.BlockSpec((P, cin, HW), lambda g: (g, 0, 0)),
            pl.BlockSpec((cin, cr3), full),
            pl.BlockSpec((1, cr3), full),
            pl.BlockSpec((cin, cr5), full),
            pl.BlockSpec((1, cr5), full),
            pl.BlockSpec((cin, c1), full),
            pl.BlockSpec((1, c1), full),
            pl.BlockSpec((9 * cr3, c3), full),
            pl.BlockSpec((1, c3), full),
            pl.BlockSpec((25 * cr5, c5), full),
            pl.BlockSpec((1, c5), full),
            pl.BlockSpec((cin, cpool), full),
            pl.BlockSpec((1, cpool), full),
        ],
        out_specs=pl.BlockSpec((P, ctot, HW), lambda g: (g, 0, 0)),
        scratch_shapes=[
            pltpu.VMEM((cin, hwp), jnp.bfloat16),
            pltpu.VMEM((cin, hwp), jnp.bfloat16),
            pltpu.VMEM((rr, hwp), f32),
            pltpu.VMEM((9 * cr3, L), f32),
            pltpu.VMEM((25 * cr5, L), f32),
            pltpu.VMEM((ctot, L), f32),
        ],
        compiler_params=pltpu.CompilerParams(
            dimension_semantics=("parallel",),
            vmem_limit_bytes=64 << 20),
    )(x3, w3r, b3r, w5r, b5r, w1, b1, w3f, b3, w5f, b5, wmp, bmp)

    return out.reshape(N, ctot, H, W)


kernel = jax.jit(_inception_fused)
